# baseline (speedup 1.0000x reference)
import functools

import jax
import jax.numpy as jnp
from jax import lax
from jax.experimental import pallas as pl
from jax.experimental.pallas import tpu as pltpu
from jax.experimental.pallas import tpu_sc as plsc

_NC = 2
_NS = 16
_LANES = 16
_UNROLL = 8
_NSPLIT = 4


def _leaky(x):
    return jnp.where(x >= 0, x, 0.2 * x)


def _pre_kernel(nf_ref, ef_ref, wn_ref, we_ref, asrc_ref, adst_ref,
                aedge_ref, h_ref, st_ref, se_ref):
    H = wn_ref.shape[0]
    NO = wn_ref.shape[2]
    nf = nf_ref[0]
    ef = ef_ref[0]
    wn_cat = jnp.concatenate([wn_ref[h] for h in range(H)], axis=1)
    hb_all = jnp.dot(nf, wn_cat, preferred_element_type=jnp.float32)
    wsrc_rows = []
    wdst_rows = []
    we_rows = []
    for h in range(H):
        h_ref[0, h] = hb_all[:, h * NO:(h + 1) * NO]
        wsrc_rows.append(lax.dot_general(
            asrc_ref[h:h + 1, :], wn_ref[h], (((1,), (1,)), ((), ())),
            preferred_element_type=jnp.float32))
        wdst_rows.append(lax.dot_general(
            adst_ref[h:h + 1, :], wn_ref[h], (((1,), (1,)), ((), ())),
            preferred_element_type=jnp.float32))
        we_rows.append(lax.dot_general(
            aedge_ref[h:h + 1, :], we_ref[h], (((1,), (1,)), ((), ())),
            preferred_element_type=jnp.float32))
    wsrc = jnp.concatenate(wsrc_rows, axis=0)
    wdst = jnp.concatenate(wdst_rows, axis=0)
    wem = jnp.concatenate(we_rows, axis=0)
    ssrc = lax.dot_general(wsrc, nf, (((1,), (1,)), ((), ())),
                           preferred_element_type=jnp.float32)
    sdst = lax.dot_general(wdst, nf, (((1,), (1,)), ((), ())),
                           preferred_element_type=jnp.float32)
    N = nf.shape[0]
    st_ref[0, :, 0:1, :] = ssrc.reshape(H, 1, N)
    st_ref[0, :, 1:2, :] = sdst.reshape(H, 1, N)
    se_ref[0] = lax.dot_general(wem, ef, (((1,), (1,)), ((), ())),
                                preferred_element_type=jnp.float32)


def _make_sc_edge_kernel(TC_CHUNK, OFF, H, N, E):
    TASKS_PER_W = (TC_CHUNK * H) // (_NC * _NS)
    NCHUNK = E // _LANES
    mesh = plsc.VectorSubcoreMesh(core_axis_name="c", subcore_axis_name="s",
                                  num_cores=_NC, num_subcores=_NS)

    @functools.partial(
        pl.kernel,
        mesh=mesh,
        compiler_params=pltpu.CompilerParams(needs_layout_passes=False),
        out_type=jax.ShapeDtypeStruct((TC_CHUNK, H, N, N), jnp.float32),
        scratch_types=[
            pltpu.VMEM((2 * E,), jnp.int32),
            pltpu.VMEM((2 * N,), jnp.float32),
            pltpu.VMEM((E,), jnp.float32),
            pltpu.VMEM((N, N), jnp.float32),
        ],
    )
    def sc_edge(sd_hbm, st_hbm, se_hbm, a_out, sd_v, st_v, se_v, a_v):
        wid = lax.axis_index("s") * _NC + lax.axis_index("c")
        base = wid * TASKS_PER_W
        z16 = jnp.zeros((_LANES,), jnp.float32)
        nn16 = jnp.full((_LANES,), N, jnp.int32)

        def zrow(r, carry):
            for cc in range(N // _LANES):
                a_v[r, pl.ds(cc * _LANES, _LANES)] = z16
            return carry

        lax.fori_loop(jnp.int32(0), jnp.int32(N), zrow, jnp.int32(0))

        for k in range(TASKS_PER_W):
            task = base + k
            tl = task // H
            h = task - tl * H
            if k == 0:
                pltpu.sync_copy(sd_hbm.at[OFF + tl], sd_v)
            pltpu.sync_copy(st_hbm.at[OFF + tl, h], st_v)
            pltpu.sync_copy(se_hbm.at[OFF + tl, h], se_v)

            def passA(i, carry):
                for u in range(_UNROLL):
                    o = (i * _UNROLL + u) * _LANES
                    s_idx = sd_v[pl.ds(o, _LANES)]
                    d_idx = sd_v[pl.ds(E + o, _LANES)]
                    v = (plsc.load_gather(st_v, [s_idx])
                         + plsc.load_gather(st_v, [d_idx + nn16])
                         + se_v[pl.ds(o, _LANES)])
                    lg = jnp.where(v >= 0, v, 0.2 * v)
                    ex = jnp.exp(jnp.minimum(lg, 75.0))
                    plsc.addupdate_scatter(a_v, [d_idx, s_idx], ex)
                return carry

            lax.fori_loop(jnp.int32(0), jnp.int32(NCHUNK // _UNROLL), passA,
                          jnp.int32(0))
            pltpu.sync_copy(a_v, a_out.at[tl, h])

            def passZ(i, carry):
                for u in range(_UNROLL):
                    o = (i * _UNROLL + u) * _LANES
                    plsc.store_scatter(
                        a_v, [sd_v[pl.ds(E + o, _LANES)],
                              sd_v[pl.ds(o, _LANES)]], z16)
                return carry

            lax.fori_loop(jnp.int32(0), jnp.int32(NCHUNK // _UNROLL), passZ,
                          jnp.int32(0))

    return sc_edge


def _agg_kernel(a_ref, h_ref, out_ref):
    H = h_ref.shape[1]
    N, NO = h_ref.shape[2], h_ref.shape[3]
    ones_col = jnp.ones((N, 1), jnp.float32)
    acc = jnp.zeros((N, NO), jnp.float32)
    for h in range(H):
        a = a_ref[0, h]
        agg = jnp.dot(a, h_ref[0, h],
                      preferred_element_type=jnp.float32)
        den = jnp.dot(a, ones_col, preferred_element_type=jnp.float32)
        rec = 1.0 / (den + 1e-16)
        acc = acc + agg * rec
    out_ref[0] = _leaky(acc * (1.0 / H))


def _gi_kernel(xs_ref, wih_ref, bih_ref, out_ref):
    k = pl.program_id(0)

    @pl.when(k == 0)
    def _():
        out_ref[...] = jnp.broadcast_to(bih_ref[...], out_ref.shape)

    out_ref[...] += lax.dot_general(
        xs_ref[...], wih_ref[...], (((1,), (1,)), ((), ())),
        preferred_element_type=jnp.float32)


def _gru_kernel(gi_ref, whh_ref, bhh_ref, out_ref):
    T = gi_ref.shape[0]
    GH = whh_ref.shape[1]

    def step(t, hprev):
        gi_t = gi_ref[pl.ds(t, 1), :]
        gh = lax.dot_general(hprev, whh_ref[...], (((1,), (1,)), ((), ())),
                             preferred_element_type=jnp.float32) + bhh_ref[...]
        i_r = gi_t[:, 0:GH]
        i_z = gi_t[:, GH:2 * GH]
        i_n = gi_t[:, 2 * GH:3 * GH]
        h_r = gh[:, 0:GH]
        h_z = gh[:, GH:2 * GH]
        h_n = gh[:, 2 * GH:3 * GH]
        r = jax.nn.sigmoid(i_r + h_r)
        z = jax.nn.sigmoid(i_z + h_z)
        n = jnp.tanh(i_n + r * h_n)
        return (1.0 - z) * n + z * hprev

    out_ref[...] = lax.fori_loop(jnp.int32(0), jnp.int32(T), step,
                                 jnp.zeros((1, GH), jnp.float32))


def _ffn_kernel(h_ref, wffn_ref, bffn_ref, out_ref):
    out_ref[...] = jax.nn.sigmoid(
        jnp.dot(h_ref[...], wffn_ref[...], preferred_element_type=jnp.float32)
        + bffn_ref[...])


def kernel(edges, node_fts, edge_fts, graph_fts, adj, W_node, W_edge, a_src,
           a_dst, a_edge, W_ih, W_hh, b_ih, b_hh, W_ffn, b_ffn):
    with jax.enable_x64(False):
        return _kernel_impl(edges, node_fts, edge_fts, graph_fts, adj, W_node,
                            W_edge, a_src, a_dst, a_edge, W_ih, W_hh, b_ih,
                            b_hh, W_ffn, b_ffn)


def _kernel_impl(edges, node_fts, edge_fts, graph_fts, adj, W_node, W_edge,
                 a_src, a_dst, a_edge, W_ih, W_hh, b_ih, b_hh, W_ffn, b_ffn):
    T, N, NODE_IN = node_fts.shape
    E = edge_fts.shape[1]
    EDGE_IN = edge_fts.shape[2]
    H, _, NO = W_node.shape
    GH = W_hh.shape[1]
    V = W_ffn.shape[1]

    sd = edges.reshape(T, 2 * E).astype(jnp.int32)

    h_all, st, s_e = pl.pallas_call(
        _pre_kernel,
        name='gn_pre',
        grid=(T,),
        in_specs=[
            pl.BlockSpec((1, N, NODE_IN), lambda t: (t, 0, 0)),
            pl.BlockSpec((1, E, EDGE_IN), lambda t: (t, 0, 0)),
            pl.BlockSpec((H, NODE_IN, NO), lambda t: (0, 0, 0)),
            pl.BlockSpec(W_edge.shape, lambda t: (0, 0, 0)),
            pl.BlockSpec(a_src.shape, lambda t: (0, 0)),
            pl.BlockSpec(a_dst.shape, lambda t: (0, 0)),
            pl.BlockSpec(a_edge.shape, lambda t: (0, 0)),
        ],
        out_specs=[
            pl.BlockSpec((1, H, N, NO), lambda t: (t, 0, 0, 0)),
            pl.BlockSpec((1, H, 2, N), lambda t: (t, 0, 0, 0)),
            pl.BlockSpec((1, H, E), lambda t: (t, 0, 0)),
        ],
        out_shape=[
            jax.ShapeDtypeStruct((T, H, N, NO), jnp.float32),
            jax.ShapeDtypeStruct((T, H, 2, N), jnp.float32),
            jax.ShapeDtypeStruct((T, H, E), jnp.float32),
        ],
    )(node_fts, edge_fts, W_node, W_edge, a_src, a_dst, a_edge)

    st2 = st.reshape(T, H, 2 * N)

    TC_CHUNK = T // _NSPLIT
    xs_chunks = []
    for c in range(_NSPLIT):
        off = c * TC_CHUNK
        sc_edge = _make_sc_edge_kernel(TC_CHUNK, off, H, N, E)
        a_mat = sc_edge(sd, st2, s_e)

        xs_c = pl.pallas_call(
            _agg_kernel,
            name=f'gn_agg{c}',
            grid=(TC_CHUNK,),
            in_specs=[
                pl.BlockSpec((1, H, N, N), lambda t: (t, 0, 0, 0)),
                pl.BlockSpec((1, H, N, NO),
                             lambda t, _o=off: (t + _o, 0, 0, 0)),
            ],
            out_specs=pl.BlockSpec((1, N, NO), lambda t: (t, 0, 0)),
            out_shape=jax.ShapeDtypeStruct((TC_CHUNK, N, NO), jnp.float32),
        )(a_mat, h_all)
        xs_chunks.append(xs_c)

    xs = jnp.concatenate(xs_chunks, axis=0)
    xs2 = xs.reshape(T, N * NO)

    K = N * NO
    KC = 2048
    gi = pl.pallas_call(
        _gi_kernel,
        name='gi',
        grid=(K // KC,),
        in_specs=[
            pl.BlockSpec((T, KC), lambda k: (0, k)),
            pl.BlockSpec((3 * GH, KC), lambda k: (0, k)),
            pl.BlockSpec((1, 3 * GH), lambda k: (0, 0)),
        ],
        out_specs=pl.BlockSpec((T, 3 * GH), lambda k: (0, 0)),
        out_shape=jax.ShapeDtypeStruct((T, 3 * GH), jnp.float32),
    )(xs2, W_ih, b_ih.reshape(1, 3 * GH))

    hT = pl.pallas_call(
        _gru_kernel,
        name='gru',
        in_specs=[
            pl.BlockSpec((T, 3 * GH), lambda: (0, 0)),
            pl.BlockSpec((3 * GH, GH), lambda: (0, 0)),
            pl.BlockSpec((1, 3 * GH), lambda: (0, 0)),
        ],
        out_specs=pl.BlockSpec((1, GH), lambda: (0, 0)),
        out_shape=jax.ShapeDtypeStruct((1, GH), jnp.float32),
    )(gi, W_hh, b_hh.reshape(1, 3 * GH))

    VC = 4096
    out = pl.pallas_call(
        _ffn_kernel,
        name='ffn',
        grid=(V // VC,),
        in_specs=[
            pl.BlockSpec((1, GH), lambda v: (0, 0)),
            pl.BlockSpec((GH, VC), lambda v: (0, v)),
            pl.BlockSpec((1, VC), lambda v: (0, v)),
        ],
        out_specs=pl.BlockSpec((1, VC), lambda v: (0, v)),
        out_shape=jax.ShapeDtypeStruct((1, V), jnp.float32),
    )(hT, W_ffn, b_ffn.reshape(1, V))

    return out


# speedup vs baseline: 1.0084x; 1.0084x over previous
import functools

import jax
import jax.numpy as jnp
from jax import lax
from jax.experimental import pallas as pl
from jax.experimental.pallas import tpu as pltpu
from jax.experimental.pallas import tpu_sc as plsc

_NC = 2
_NS = 16
_LANES = 16
_UNROLL = 8
_NSPLIT = 2


def _leaky(x):
    return jnp.where(x >= 0, x, 0.2 * x)


def _pre_kernel(nf_ref, ef_ref, wn_ref, we_ref, asrc_ref, adst_ref,
                aedge_ref, h_ref, st_ref, se_ref,
                wncat_s, wsrc_s, wdst_s, wem_s):
    H = wn_ref.shape[0]
    NO = wn_ref.shape[2]
    nf = nf_ref[0]
    ef = ef_ref[0]

    @pl.when(pl.program_id(0) == 0)
    def _():
        wncat_s[...] = jnp.concatenate([wn_ref[h] for h in range(H)], axis=1)
        wsrc_rows = []
        wdst_rows = []
        we_rows = []
        for h in range(H):
            wsrc_rows.append(lax.dot_general(
                asrc_ref[h:h + 1, :], wn_ref[h], (((1,), (1,)), ((), ())),
                preferred_element_type=jnp.float32))
            wdst_rows.append(lax.dot_general(
                adst_ref[h:h + 1, :], wn_ref[h], (((1,), (1,)), ((), ())),
                preferred_element_type=jnp.float32))
            we_rows.append(lax.dot_general(
                aedge_ref[h:h + 1, :], we_ref[h], (((1,), (1,)), ((), ())),
                preferred_element_type=jnp.float32))
        wsrc_s[...] = jnp.concatenate(wsrc_rows, axis=0)
        wdst_s[...] = jnp.concatenate(wdst_rows, axis=0)
        wem_s[...] = jnp.concatenate(we_rows, axis=0)

    hb_all = jnp.dot(nf, wncat_s[...], preferred_element_type=jnp.float32)
    for h in range(H):
        h_ref[0, h] = hb_all[:, h * NO:(h + 1) * NO]
    ssrc = lax.dot_general(wsrc_s[...], nf, (((1,), (1,)), ((), ())),
                           preferred_element_type=jnp.float32)
    sdst = lax.dot_general(wdst_s[...], nf, (((1,), (1,)), ((), ())),
                           preferred_element_type=jnp.float32)
    N = nf.shape[0]
    st_ref[0, :, 0:1, :] = ssrc.reshape(H, 1, N)
    st_ref[0, :, 1:2, :] = sdst.reshape(H, 1, N)
    se_ref[0] = lax.dot_general(wem_s[...], ef, (((1,), (1,)), ((), ())),
                                preferred_element_type=jnp.float32)


def _make_sc_edge_kernel(TC_CHUNK, OFF, H, N, E):
    TASKS_PER_W = (TC_CHUNK * H) // (_NC * _NS)
    NCHUNK = E // _LANES
    mesh = plsc.VectorSubcoreMesh(core_axis_name="c", subcore_axis_name="s",
                                  num_cores=_NC, num_subcores=_NS)

    @functools.partial(
        pl.kernel,
        mesh=mesh,
        compiler_params=pltpu.CompilerParams(needs_layout_passes=False),
        out_type=jax.ShapeDtypeStruct((TC_CHUNK, H, N, N), jnp.float32),
        scratch_types=[
            pltpu.VMEM((2 * E,), jnp.int32),
            pltpu.VMEM((2 * N,), jnp.float32),
            pltpu.VMEM((E,), jnp.float32),
            pltpu.VMEM((N, N), jnp.float32),
        ],
    )
    def sc_edge(sd_hbm, st_hbm, se_hbm, a_out, sd_v, st_v, se_v, a_v):
        wid = lax.axis_index("s") * _NC + lax.axis_index("c")
        base = wid * TASKS_PER_W
        z16 = jnp.zeros((_LANES,), jnp.float32)
        nn16 = jnp.full((_LANES,), N, jnp.int32)

        def zrow(r, carry):
            for cc in range(N // _LANES):
                a_v[r, pl.ds(cc * _LANES, _LANES)] = z16
            return carry

        lax.fori_loop(jnp.int32(0), jnp.int32(N), zrow, jnp.int32(0))

        for k in range(TASKS_PER_W):
            task = base + k
            tl = task // H
            h = task - tl * H
            if k == 0:
                pltpu.sync_copy(sd_hbm.at[OFF + tl], sd_v)
            pltpu.sync_copy(st_hbm.at[OFF + tl, h], st_v)
            pltpu.sync_copy(se_hbm.at[OFF + tl, h], se_v)

            def passA(i, carry):
                for u in range(_UNROLL):
                    o = (i * _UNROLL + u) * _LANES
                    s_idx = sd_v[pl.ds(o, _LANES)]
                    d_idx = sd_v[pl.ds(E + o, _LANES)]
                    v = (plsc.load_gather(st_v, [s_idx])
                         + plsc.load_gather(st_v, [d_idx + nn16])
                         + se_v[pl.ds(o, _LANES)])
                    lg = jnp.where(v >= 0, v, 0.2 * v)
                    ex = jnp.exp(jnp.minimum(lg, 75.0))
                    plsc.addupdate_scatter(a_v, [d_idx, s_idx], ex)
                return carry

            lax.fori_loop(jnp.int32(0), jnp.int32(NCHUNK // _UNROLL), passA,
                          jnp.int32(0))
            pltpu.sync_copy(a_v, a_out.at[tl, h])

            def passZ(i, carry):
                for u in range(_UNROLL):
                    o = (i * _UNROLL + u) * _LANES
                    plsc.store_scatter(
                        a_v, [sd_v[pl.ds(E + o, _LANES)],
                              sd_v[pl.ds(o, _LANES)]], z16)
                return carry

            lax.fori_loop(jnp.int32(0), jnp.int32(NCHUNK // _UNROLL), passZ,
                          jnp.int32(0))

    return sc_edge


def _agg_kernel(a_ref, h_ref, out_ref):
    H = h_ref.shape[1]
    N, NO = h_ref.shape[2], h_ref.shape[3]
    ones_col = jnp.ones((N, 1), jnp.float32)
    acc = jnp.zeros((N, NO), jnp.float32)
    for h in range(H):
        a = a_ref[0, h]
        agg = jnp.dot(a, h_ref[0, h],
                      preferred_element_type=jnp.float32)
        den = jnp.dot(a, ones_col, preferred_element_type=jnp.float32)
        rec = 1.0 / (den + 1e-16)
        acc = acc + agg * rec
    out_ref[0] = _leaky(acc * (1.0 / H))


def _make_tail_kernel(T, K, GH, V, KC, VC):
    NK = K // KC
    NV = V // VC

    def tail(xs_ref, wih_hbm, bih_ref, whh_ref, bhh_ref, wffn_hbm,
             bffn_ref, out_ref, wih_b0, wih_b1, wffn_b0, wffn_b1, gi_s,
             sem_ih0, sem_ih1, sem_f0, sem_f1):
        wih_bufs = (wih_b0, wih_b1)
        sem_ih = (sem_ih0, sem_ih1)
        wffn_bufs = (wffn_b0, wffn_b1)
        sem_f = (sem_f0, sem_f1)

        def ih_copy(k):
            return pltpu.make_async_copy(
                wih_hbm.at[:, pl.ds(k * KC, KC)], wih_bufs[k % 2],
                sem_ih[k % 2])

        def f_copy(v):
            return pltpu.make_async_copy(
                wffn_hbm.at[:, pl.ds(v * VC, VC)], wffn_bufs[v % 2],
                sem_f[v % 2])

        ih_copy(0).start()
        f_copy(0).start()
        f_copy(1).start()

        gi_s[...] = jnp.broadcast_to(bih_ref[...], (T, 3 * GH))
        for k in range(NK):
            if k + 1 < NK:
                ih_copy(k + 1).start()
            ih_copy(k).wait()
            gi_s[...] += lax.dot_general(
                xs_ref[:, pl.ds(k * KC, KC)], wih_bufs[k % 2][...],
                (((1,), (1,)), ((), ())), preferred_element_type=jnp.float32)

        def step(t, hprev):
            gi_t = gi_s[pl.ds(t, 1), :]
            gh = lax.dot_general(
                hprev, whh_ref[...], (((1,), (1,)), ((), ())),
                preferred_element_type=jnp.float32) + bhh_ref[...]
            i_r = gi_t[:, 0:GH]
            i_z = gi_t[:, GH:2 * GH]
            i_n = gi_t[:, 2 * GH:3 * GH]
            h_r = gh[:, 0:GH]
            h_z = gh[:, GH:2 * GH]
            h_n = gh[:, 2 * GH:3 * GH]
            r = jax.nn.sigmoid(i_r + h_r)
            z = jax.nn.sigmoid(i_z + h_z)
            n = jnp.tanh(i_n + r * h_n)
            return (1.0 - z) * n + z * hprev

        hT = lax.fori_loop(jnp.int32(0), jnp.int32(T), step,
                           jnp.zeros((1, GH), jnp.float32))

        for v in range(NV):
            f_copy(v).wait()
            sl = pl.ds(v * VC, VC)
            out_ref[:, sl] = jax.nn.sigmoid(
                jnp.dot(hT, wffn_bufs[v % 2][...],
                        preferred_element_type=jnp.float32)
                + bffn_ref[:, sl])
            if v + 2 < NV:
                f_copy(v + 2).start()

    return tail


def kernel(edges, node_fts, edge_fts, graph_fts, adj, W_node, W_edge, a_src,
           a_dst, a_edge, W_ih, W_hh, b_ih, b_hh, W_ffn, b_ffn):
    with jax.enable_x64(False):
        return _kernel_impl(edges, node_fts, edge_fts, graph_fts, adj, W_node,
                            W_edge, a_src, a_dst, a_edge, W_ih, W_hh, b_ih,
                            b_hh, W_ffn, b_ffn)


def _kernel_impl(edges, node_fts, edge_fts, graph_fts, adj, W_node, W_edge,
                 a_src, a_dst, a_edge, W_ih, W_hh, b_ih, b_hh, W_ffn, b_ffn):
    T, N, NODE_IN = node_fts.shape
    E = edge_fts.shape[1]
    EDGE_IN = edge_fts.shape[2]
    H, _, NO = W_node.shape
    GH = W_hh.shape[1]
    V = W_ffn.shape[1]

    sd = edges.reshape(T, 2 * E).astype(jnp.int32)

    h_all, st, s_e = pl.pallas_call(
        _pre_kernel,
        name='gn_pre',
        grid=(T,),
        in_specs=[
            pl.BlockSpec((1, N, NODE_IN), lambda t: (t, 0, 0)),
            pl.BlockSpec((1, E, EDGE_IN), lambda t: (t, 0, 0)),
            pl.BlockSpec((H, NODE_IN, NO), lambda t: (0, 0, 0)),
            pl.BlockSpec(W_edge.shape, lambda t: (0, 0, 0)),
            pl.BlockSpec(a_src.shape, lambda t: (0, 0)),
            pl.BlockSpec(a_dst.shape, lambda t: (0, 0)),
            pl.BlockSpec(a_edge.shape, lambda t: (0, 0)),
        ],
        out_specs=[
            pl.BlockSpec((1, H, N, NO), lambda t: (t, 0, 0, 0)),
            pl.BlockSpec((1, H, 2, N), lambda t: (t, 0, 0, 0)),
            pl.BlockSpec((1, H, E), lambda t: (t, 0, 0)),
        ],
        out_shape=[
            jax.ShapeDtypeStruct((T, H, N, NO), jnp.float32),
            jax.ShapeDtypeStruct((T, H, 2, N), jnp.float32),
            jax.ShapeDtypeStruct((T, H, E), jnp.float32),
        ],
        scratch_shapes=[
            pltpu.VMEM((NODE_IN, H * NO), jnp.float32),
            pltpu.VMEM((H, NODE_IN), jnp.float32),
            pltpu.VMEM((H, NODE_IN), jnp.float32),
            pltpu.VMEM((H, EDGE_IN), jnp.float32),
        ],
    )(node_fts, edge_fts, W_node, W_edge, a_src, a_dst, a_edge)

    st2 = st.reshape(T, H, 2 * N)

    TC_CHUNK = T // _NSPLIT
    xs_chunks = []
    for c in range(_NSPLIT):
        off = c * TC_CHUNK
        sc_edge = _make_sc_edge_kernel(TC_CHUNK, off, H, N, E)
        a_mat = sc_edge(sd, st2, s_e)

        xs_c = pl.pallas_call(
            _agg_kernel,
            name=f'gn_agg{c}',
            grid=(TC_CHUNK,),
            in_specs=[
                pl.BlockSpec((1, H, N, N), lambda t: (t, 0, 0, 0)),
                pl.BlockSpec((1, H, N, NO),
                             lambda t, _o=off: (t + _o, 0, 0, 0)),
            ],
            out_specs=pl.BlockSpec((1, N, NO), lambda t: (t, 0, 0)),
            out_shape=jax.ShapeDtypeStruct((TC_CHUNK, N, NO), jnp.float32),
        )(a_mat, h_all)
        xs_chunks.append(xs_c)

    xs = jnp.concatenate(xs_chunks, axis=0)
    xs2 = xs.reshape(T, N * NO)

    K = N * NO
    KC = 2048
    VC = 4096
    tail = _make_tail_kernel(T, K, GH, V, KC, VC)
    out = pl.pallas_call(
        tail,
        name='tail',
        in_specs=[
            pl.BlockSpec((T, K), lambda: (0, 0)),
            pl.BlockSpec(memory_space=pltpu.MemorySpace.HBM),
            pl.BlockSpec((1, 3 * GH), lambda: (0, 0)),
            pl.BlockSpec((3 * GH, GH), lambda: (0, 0)),
            pl.BlockSpec((1, 3 * GH), lambda: (0, 0)),
            pl.BlockSpec(memory_space=pltpu.MemorySpace.HBM),
            pl.BlockSpec((1, V), lambda: (0, 0)),
        ],
        out_specs=pl.BlockSpec((1, V), lambda: (0, 0)),
        out_shape=jax.ShapeDtypeStruct((1, V), jnp.float32),
        scratch_shapes=[
            pltpu.VMEM((3 * GH, KC), jnp.float32),
            pltpu.VMEM((3 * GH, KC), jnp.float32),
            pltpu.VMEM((GH, VC), jnp.float32),
            pltpu.VMEM((GH, VC), jnp.float32),
            pltpu.VMEM((T, 3 * GH), jnp.float32),
            pltpu.SemaphoreType.DMA,
            pltpu.SemaphoreType.DMA,
            pltpu.SemaphoreType.DMA,
            pltpu.SemaphoreType.DMA,
        ],
    )(xs2, W_ih, b_ih.reshape(1, 3 * GH), W_hh, b_hh.reshape(1, 3 * GH),
      W_ffn, b_ffn.reshape(1, V))

    return out


# speedup vs baseline: 1.0256x; 1.0170x over previous
import functools

import jax
import jax.numpy as jnp
from jax import lax
from jax.experimental import pallas as pl
from jax.experimental.pallas import tpu as pltpu
from jax.experimental.pallas import tpu_sc as plsc

_NC = 2
_NS = 16
_LANES = 16
_UNROLL = 8
_NSPLIT = 2


def _leaky(x):
    return jnp.where(x >= 0, x, 0.2 * x)


def _pre_kernel(nf_ref, ef_ref, wn_ref, we_ref, asrc_ref, adst_ref,
                aedge_ref, h_ref, st_ref, se_ref,
                wncat_s, wsrc_s, wdst_s, wem_s):
    H = wn_ref.shape[0]
    NO = wn_ref.shape[2]
    nf = nf_ref[0]
    ef = ef_ref[0]

    @pl.when(pl.program_id(0) == 0)
    def _():
        wncat_s[...] = jnp.concatenate([wn_ref[h] for h in range(H)], axis=1)
        wsrc_rows = []
        wdst_rows = []
        we_rows = []
        for h in range(H):
            wsrc_rows.append(lax.dot_general(
                asrc_ref[h:h + 1, :], wn_ref[h], (((1,), (1,)), ((), ())),
                preferred_element_type=jnp.float32))
            wdst_rows.append(lax.dot_general(
                adst_ref[h:h + 1, :], wn_ref[h], (((1,), (1,)), ((), ())),
                preferred_element_type=jnp.float32))
            we_rows.append(lax.dot_general(
                aedge_ref[h:h + 1, :], we_ref[h], (((1,), (1,)), ((), ())),
                preferred_element_type=jnp.float32))
        wsrc_s[...] = jnp.concatenate(wsrc_rows, axis=0)
        wdst_s[...] = jnp.concatenate(wdst_rows, axis=0)
        wem_s[...] = jnp.concatenate(we_rows, axis=0)

    hb_all = jnp.dot(nf, wncat_s[...], preferred_element_type=jnp.float32)
    for h in range(H):
        h_ref[0, h] = hb_all[:, h * NO:(h + 1) * NO]
    ssrc = lax.dot_general(wsrc_s[...], nf, (((1,), (1,)), ((), ())),
                           preferred_element_type=jnp.float32)
    sdst = lax.dot_general(wdst_s[...], nf, (((1,), (1,)), ((), ())),
                           preferred_element_type=jnp.float32)
    N = nf.shape[0]
    st_ref[0, :, 0:1, :] = ssrc.reshape(H, 1, N)
    st_ref[0, :, 1:2, :] = sdst.reshape(H, 1, N)
    se_ref[0] = lax.dot_general(wem_s[...], ef, (((1,), (1,)), ((), ())),
                                preferred_element_type=jnp.float32)


def _make_sc_edge_kernel(TC_CHUNK, OFF, H, N, E):
    TASKS_PER_W = (TC_CHUNK * H) // (_NC * _NS)
    NCHUNK = E // _LANES
    mesh = plsc.VectorSubcoreMesh(core_axis_name="c", subcore_axis_name="s",
                                  num_cores=_NC, num_subcores=_NS)

    @functools.partial(
        pl.kernel,
        mesh=mesh,
        compiler_params=pltpu.CompilerParams(needs_layout_passes=False),
        out_type=jax.ShapeDtypeStruct((TC_CHUNK, H, N, N), jnp.float32),
        scratch_types=[
            pltpu.VMEM((2 * E,), jnp.int32),
            pltpu.VMEM((2 * N,), jnp.float32),
            pltpu.VMEM((E,), jnp.float32),
            pltpu.VMEM((N, N), jnp.float32),
        ],
    )
    def sc_edge(sd_hbm, st_hbm, se_hbm, a_out, sd_v, st_v, se_v, a_v):
        wid = lax.axis_index("s") * _NC + lax.axis_index("c")
        base = wid * TASKS_PER_W
        z16 = jnp.zeros((_LANES,), jnp.float32)
        nn16 = jnp.full((_LANES,), N, jnp.int32)

        def zrow(r, carry):
            for cc in range(N // _LANES):
                a_v[r, pl.ds(cc * _LANES, _LANES)] = z16
            return carry

        lax.fori_loop(jnp.int32(0), jnp.int32(N), zrow, jnp.int32(0))

        for k in range(TASKS_PER_W):
            task = base + k
            tl = task // H
            h = task - tl * H
            if k == 0:
                pltpu.sync_copy(sd_hbm.at[OFF + tl], sd_v)
            pltpu.sync_copy(st_hbm.at[OFF + tl, h], st_v)
            pltpu.sync_copy(se_hbm.at[OFF + tl, h], se_v)

            def passA(i, carry):
                for u in range(_UNROLL):
                    o = (i * _UNROLL + u) * _LANES
                    s_idx = sd_v[pl.ds(o, _LANES)]
                    d_idx = sd_v[pl.ds(E + o, _LANES)]
                    v = (plsc.load_gather(st_v, [s_idx])
                         + plsc.load_gather(st_v, [d_idx + nn16])
                         + se_v[pl.ds(o, _LANES)])
                    lg = jnp.where(v >= 0, v, 0.2 * v)
                    ex = jnp.exp(jnp.minimum(lg, 75.0))
                    plsc.addupdate_scatter(a_v, [d_idx, s_idx], ex)
                return carry

            lax.fori_loop(jnp.int32(0), jnp.int32(NCHUNK // _UNROLL), passA,
                          jnp.int32(0))
            pltpu.sync_copy(a_v, a_out.at[tl, h])

            def passZ(i, carry):
                for u in range(_UNROLL):
                    o = (i * _UNROLL + u) * _LANES
                    plsc.store_scatter(
                        a_v, [sd_v[pl.ds(E + o, _LANES)],
                              sd_v[pl.ds(o, _LANES)]], z16)
                return carry

            lax.fori_loop(jnp.int32(0), jnp.int32(NCHUNK // _UNROLL), passZ,
                          jnp.int32(0))

    return sc_edge


def _agg_kernel(a_ref, h_ref, out_ref):
    H = h_ref.shape[1]
    N, NO = h_ref.shape[2], h_ref.shape[3]
    ones_col = jnp.ones((N, 1), jnp.float32)
    acc = jnp.zeros((N, NO), jnp.float32)
    for h in range(H):
        a = a_ref[0, h]
        agg = jnp.dot(a, h_ref[0, h],
                      preferred_element_type=jnp.float32)
        den = jnp.dot(a, ones_col, preferred_element_type=jnp.float32)
        rec = 1.0 / (den + 1e-16)
        acc = acc + agg * rec
    out_ref[0] = _leaky(acc * (1.0 / H))


def _make_tail_kernel(T, K, GH, V, KC, VC):
    NK = K // KC
    NV = V // VC

    def tail(xs0_ref, xs1_ref, wih_hbm, bih_ref, whh_ref, bhh_ref, wffn_hbm,
             bffn_ref, out_ref, wih_b0, wih_b1, wffn_b0, wffn_b1, gi_s,
             sem_ih0, sem_ih1, sem_f0, sem_f1):
        TH = T // 2
        wih_bufs = (wih_b0, wih_b1)
        sem_ih = (sem_ih0, sem_ih1)
        wffn_bufs = (wffn_b0, wffn_b1)
        sem_f = (sem_f0, sem_f1)

        def ih_copy(k):
            return pltpu.make_async_copy(
                wih_hbm.at[:, pl.ds(k * KC, KC)], wih_bufs[k % 2],
                sem_ih[k % 2])

        def f_copy(v):
            return pltpu.make_async_copy(
                wffn_hbm.at[:, pl.ds(v * VC, VC)], wffn_bufs[v % 2],
                sem_f[v % 2])

        ih_copy(0).start()
        f_copy(0).start()
        f_copy(1).start()

        gi_s[...] = jnp.broadcast_to(bih_ref[...], (T, 3 * GH))
        for k in range(NK):
            if k + 1 < NK:
                ih_copy(k + 1).start()
            ih_copy(k).wait()
            w = wih_bufs[k % 2][...]
            gi_s[0:TH, :] += lax.dot_general(
                xs0_ref[:, pl.ds(k * KC, KC)], w,
                (((1,), (1,)), ((), ())), preferred_element_type=jnp.float32)
            gi_s[TH:T, :] += lax.dot_general(
                xs1_ref[:, pl.ds(k * KC, KC)], w,
                (((1,), (1,)), ((), ())), preferred_element_type=jnp.float32)

        def step(t, hprev):
            gi_t = gi_s[pl.ds(t, 1), :]
            gh = lax.dot_general(
                hprev, whh_ref[...], (((1,), (1,)), ((), ())),
                preferred_element_type=jnp.float32) + bhh_ref[...]
            i_r = gi_t[:, 0:GH]
            i_z = gi_t[:, GH:2 * GH]
            i_n = gi_t[:, 2 * GH:3 * GH]
            h_r = gh[:, 0:GH]
            h_z = gh[:, GH:2 * GH]
            h_n = gh[:, 2 * GH:3 * GH]
            r = jax.nn.sigmoid(i_r + h_r)
            z = jax.nn.sigmoid(i_z + h_z)
            n = jnp.tanh(i_n + r * h_n)
            return (1.0 - z) * n + z * hprev

        hT = lax.fori_loop(jnp.int32(0), jnp.int32(T), step,
                           jnp.zeros((1, GH), jnp.float32))

        for v in range(NV):
            f_copy(v).wait()
            sl = pl.ds(v * VC, VC)
            out_ref[:, sl] = jax.nn.sigmoid(
                jnp.dot(hT, wffn_bufs[v % 2][...],
                        preferred_element_type=jnp.float32)
                + bffn_ref[:, sl])
            if v + 2 < NV:
                f_copy(v + 2).start()

    return tail


def kernel(edges, node_fts, edge_fts, graph_fts, adj, W_node, W_edge, a_src,
           a_dst, a_edge, W_ih, W_hh, b_ih, b_hh, W_ffn, b_ffn):
    with jax.enable_x64(False):
        return _kernel_impl(edges, node_fts, edge_fts, graph_fts, adj, W_node,
                            W_edge, a_src, a_dst, a_edge, W_ih, W_hh, b_ih,
                            b_hh, W_ffn, b_ffn)


def _kernel_impl(edges, node_fts, edge_fts, graph_fts, adj, W_node, W_edge,
                 a_src, a_dst, a_edge, W_ih, W_hh, b_ih, b_hh, W_ffn, b_ffn):
    T, N, NODE_IN = node_fts.shape
    E = edge_fts.shape[1]
    EDGE_IN = edge_fts.shape[2]
    H, _, NO = W_node.shape
    GH = W_hh.shape[1]
    V = W_ffn.shape[1]

    sd = edges.reshape(T, 2 * E).astype(jnp.int32)

    h_all, st, s_e = pl.pallas_call(
        _pre_kernel,
        name='gn_pre',
        grid=(T,),
        in_specs=[
            pl.BlockSpec((1, N, NODE_IN), lambda t: (t, 0, 0)),
            pl.BlockSpec((1, E, EDGE_IN), lambda t: (t, 0, 0)),
            pl.BlockSpec((H, NODE_IN, NO), lambda t: (0, 0, 0)),
            pl.BlockSpec(W_edge.shape, lambda t: (0, 0, 0)),
            pl.BlockSpec(a_src.shape, lambda t: (0, 0)),
            pl.BlockSpec(a_dst.shape, lambda t: (0, 0)),
            pl.BlockSpec(a_edge.shape, lambda t: (0, 0)),
        ],
        out_specs=[
            pl.BlockSpec((1, H, N, NO), lambda t: (t, 0, 0, 0)),
            pl.BlockSpec((1, H, 2, N), lambda t: (t, 0, 0, 0)),
            pl.BlockSpec((1, H, E), lambda t: (t, 0, 0)),
        ],
        out_shape=[
            jax.ShapeDtypeStruct((T, H, N, NO), jnp.float32),
            jax.ShapeDtypeStruct((T, H, 2, N), jnp.float32),
            jax.ShapeDtypeStruct((T, H, E), jnp.float32),
        ],
        scratch_shapes=[
            pltpu.VMEM((NODE_IN, H * NO), jnp.float32),
            pltpu.VMEM((H, NODE_IN), jnp.float32),
            pltpu.VMEM((H, NODE_IN), jnp.float32),
            pltpu.VMEM((H, EDGE_IN), jnp.float32),
        ],
    )(node_fts, edge_fts, W_node, W_edge, a_src, a_dst, a_edge)

    st2 = st.reshape(T, H, 2 * N)

    TC_CHUNK = T // _NSPLIT
    xs_chunks = []
    for c in range(_NSPLIT):
        off = c * TC_CHUNK
        sc_edge = _make_sc_edge_kernel(TC_CHUNK, off, H, N, E)
        a_mat = sc_edge(sd, st2, s_e)

        xs_c = pl.pallas_call(
            _agg_kernel,
            name=f'gn_agg{c}',
            grid=(TC_CHUNK,),
            in_specs=[
                pl.BlockSpec((1, H, N, N), lambda t: (t, 0, 0, 0)),
                pl.BlockSpec((1, H, N, NO),
                             lambda t, _o=off: (t + _o, 0, 0, 0)),
            ],
            out_specs=pl.BlockSpec((1, N, NO), lambda t: (t, 0, 0)),
            out_shape=jax.ShapeDtypeStruct((TC_CHUNK, N, NO), jnp.float32),
        )(a_mat, h_all)
        xs_chunks.append(xs_c)

    K = N * NO
    TH = T // 2
    xs0 = xs_chunks[0].reshape(TH, K)
    xs1 = xs_chunks[1].reshape(TH, K)

    KC = 2048
    VC = 4096
    tail = _make_tail_kernel(T, K, GH, V, KC, VC)
    out = pl.pallas_call(
        tail,
        name='tail',
        in_specs=[
            pl.BlockSpec((TH, K), lambda: (0, 0)),
            pl.BlockSpec((TH, K), lambda: (0, 0)),
            pl.BlockSpec(memory_space=pltpu.MemorySpace.HBM),
            pl.BlockSpec((1, 3 * GH), lambda: (0, 0)),
            pl.BlockSpec((3 * GH, GH), lambda: (0, 0)),
            pl.BlockSpec((1, 3 * GH), lambda: (0, 0)),
            pl.BlockSpec(memory_space=pltpu.MemorySpace.HBM),
            pl.BlockSpec((1, V), lambda: (0, 0)),
        ],
        out_specs=pl.BlockSpec((1, V), lambda: (0, 0)),
        out_shape=jax.ShapeDtypeStruct((1, V), jnp.float32),
        scratch_shapes=[
            pltpu.VMEM((3 * GH, KC), jnp.float32),
            pltpu.VMEM((3 * GH, KC), jnp.float32),
            pltpu.VMEM((GH, VC), jnp.float32),
            pltpu.VMEM((GH, VC), jnp.float32),
            pltpu.VMEM((T, 3 * GH), jnp.float32),
            pltpu.SemaphoreType.DMA,
            pltpu.SemaphoreType.DMA,
            pltpu.SemaphoreType.DMA,
            pltpu.SemaphoreType.DMA,
        ],
    )(xs0, xs1, W_ih, b_ih.reshape(1, 3 * GH), W_hh, b_hh.reshape(1, 3 * GH),
      W_ffn, b_ffn.reshape(1, V))

    return out
